# baseline (speedup 1.0000x reference)
import functools

import jax
import jax.numpy as jnp
from jax import lax
from jax.experimental import pallas as pl
from jax.experimental.pallas import tpu as pltpu
from jax.experimental.pallas import tpu_sc as plsc

B = 4096
L1, L2, L3 = 20, 200, 26
L1P, L3P = 24, 32
H = 128
NHID = 256

NC, NS = 2, 16
NW = NC * NS
CH = 128
NBUF = 4

N1, N2, N3 = B * L1P, B * L2, B * L3P
P1, P2, P3 = N1 // NW, N2 // NW, N3 // NW


def _sc_gather():
    mesh = plsc.VectorSubcoreMesh(core_axis_name="c", subcore_axis_name="s")

    @functools.partial(
        pl.kernel,
        mesh=mesh,
        out_type=(
            jax.ShapeDtypeStruct((N1, H), jnp.float32),
            jax.ShapeDtypeStruct((N2, H), jnp.float32),
            jax.ShapeDtypeStruct((N3, H), jnp.float32),
        ),
        scratch_types=[
            pltpu.VMEM((P1,), jnp.int32),
            pltpu.VMEM((P2,), jnp.int32),
            pltpu.VMEM((P3,), jnp.int32),
            pltpu.VMEM((NBUF, CH, H), jnp.float32),
            pltpu.SemaphoreType.DMA((NBUF,)),
            pltpu.SemaphoreType.DMA((NBUF,)),
        ],
    )
    def k(idx1, idx2, idx3, t1, t2, t3, o1, o2, o3,
          idx1_v, idx2_v, idx3_v, rows_v, gsem, wsem):
        wid = lax.axis_index("s") * NC + lax.axis_index("c")

        def run(idx_hbm, idx_v, table_hbm, out_hbm, per_worker):
            n = per_worker // CH
            base = wid * per_worker
            pltpu.sync_copy(idx_hbm.at[pl.ds(base, per_worker)], idx_v)

            def gth(c, b):
                return pltpu.make_async_copy(
                    table_hbm.at[idx_v.at[pl.ds(c * CH, CH)]],
                    rows_v.at[b], gsem.at[b])

            def wb(c, b):
                return pltpu.make_async_copy(
                    rows_v.at[b], out_hbm.at[pl.ds(base + c * CH, CH)],
                    wsem.at[b])

            for b in range(NBUF):
                gth(b, b).start()

            m4 = ((n - NBUF) // NBUF) * NBUF

            def body(i, _):
                for b in range(NBUF):
                    c = i * NBUF + b
                    gth(c, b).wait()
                    wb(c, b).start()
                    wb(c, b).wait()
                    gth(c + NBUF, b).start()
                return 0

            lax.fori_loop(0, m4 // NBUF, body, 0)

            for cc in range(m4, n):
                b = cc % NBUF
                gth(cc, b).wait()
                wb(cc, b).start()
                wb(cc, b).wait()
                if cc + NBUF < n:
                    gth(cc + NBUF, b).start()

        run(idx1, idx1_v, t1, o1, P1)
        run(idx2, idx2_v, t2, o2, P2)
        run(idx3, idx3_v, t3, o3, P3)

    return k


_LB = 8
_MB = 1024
_NK1, _NK2, _NK3 = L1P // _LB, L2 // _LB, L3P // _LB
_NK = _NK1 + _NK2 + _NK3


def _tc_body(g1, g2, g3, w1, w2, w3, bi, wf, bf, out_ref, acc_ref):
    k = pl.program_id(1)

    @pl.when(k == 0)
    def _():
        acc_ref[...] = jnp.broadcast_to(bi[...], (_MB, NHID))

    def seg(g, w):
        for i in range(_LB):
            acc_ref[...] += jnp.dot(
                g[:, i, :].astype(jnp.bfloat16), w[i * H : (i + 1) * H, :],
                preferred_element_type=jnp.float32)

    @pl.when(k < _NK1)
    def _():
        seg(g1[...], w1[...])

    @pl.when((k >= _NK1) & (k < _NK1 + _NK2))
    def _():
        seg(g2[...], w2[...])

    @pl.when(k >= _NK1 + _NK2)
    def _():
        seg(g3[...], w3[...])

    @pl.when(k == _NK - 1)
    def _():
        r = acc_ref[...] * wf[...]
        out_ref[...] = jnp.sum(r, axis=1, keepdims=True) + bf[0, 0]


def _tc_mlp(g1, g2, g3, w1, w2, w3, b_inter, w_final, b_final):
    def g_spec(lo, nk):
        return pl.BlockSpec(
            (_MB, _LB, H),
            lambda m, k: (m, jnp.clip(k - lo, 0, nk - 1), 0),
        )

    def w_spec(lo, nk):
        return pl.BlockSpec(
            (_LB * H, NHID),
            lambda m, k: (jnp.clip(k - lo, 0, nk - 1), 0),
        )

    return pl.pallas_call(
        _tc_body,
        grid=(B // _MB, _NK),
        in_specs=[
            g_spec(0, _NK1),
            g_spec(_NK1, _NK2),
            g_spec(_NK1 + _NK2, _NK3),
            w_spec(0, _NK1),
            w_spec(_NK1, _NK2),
            w_spec(_NK1 + _NK2, _NK3),
            pl.BlockSpec((1, NHID), lambda m, k: (0, 0)),
            pl.BlockSpec((1, NHID), lambda m, k: (0, 0)),
            pl.BlockSpec(memory_space=pltpu.SMEM),
        ],
        out_specs=pl.BlockSpec((_MB, 1), lambda m, k: (m, 0)),
        out_shape=jax.ShapeDtypeStruct((B, 1), jnp.float32),
        scratch_shapes=[pltpu.VMEM((_MB, NHID), jnp.float32)],
    )(g1, g2, g3, w1, w2, w3, b_inter, w_final, b_final)


def _rearrange_w(w_seg, lt, lpad):
    wt = w_seg.reshape(NHID, H, lt).transpose(2, 1, 0).reshape(lt * H, NHID)
    if lpad != lt:
        wt = jnp.concatenate(
            [wt, jnp.zeros(((lpad - lt) * H, NHID), wt.dtype)])
    return wt


def kernel(input1, input2, input3, title_emb, full_emb, cat_emb,
           W_inter, b_inter, W_final, b_final):
    idx1 = jnp.pad(input1.astype(jnp.int32), ((0, 0), (0, L1P - L1)))
    idx3 = jnp.pad(input3.astype(jnp.int32), ((0, 0), (0, L3P - L3)))
    idx1 = idx1.reshape(-1)
    idx2 = input2.reshape(-1).astype(jnp.int32)
    idx3 = idx3.reshape(-1)

    g1, g2, g3 = _sc_gather()(idx1, idx2, idx3, title_emb, full_emb, cat_emb)

    wb = W_inter.astype(jnp.bfloat16)
    w1 = _rearrange_w(wb[:, : H * L1], L1, L1P)
    w2 = _rearrange_w(wb[:, H * L1 : H * (L1 + L2)], L2, L2)
    w3 = _rearrange_w(wb[:, H * (L1 + L2) :], L3, L3P)

    return _tc_mlp(
        g1.reshape(B, L1P, H), g2.reshape(B, L2, H), g3.reshape(B, L3P, H),
        w1, w2, w3,
        b_inter.reshape(1, NHID),
        W_final.reshape(1, NHID),
        b_final.reshape(1, 1).astype(jnp.float32),
    )


# speedup vs baseline: 2.5366x; 2.5366x over previous
import functools

import jax
import jax.numpy as jnp
from jax import lax
from jax.experimental import pallas as pl
from jax.experimental.pallas import tpu as pltpu
from jax.experimental.pallas import tpu_sc as plsc

B = 4096
L1, L2, L3 = 20, 200, 26
L1P, L3P = 24, 32
H = 128
NHID = 256

NC, NS = 2, 16
NW = NC * NS
CH = 128
NBUF = 4

N1, N2, N3 = B * L1P, B * L2, B * L3P
P1, P2, P3 = N1 // NW, N2 // NW, N3 // NW


def _sc_gather():
    mesh = plsc.VectorSubcoreMesh(core_axis_name="c", subcore_axis_name="s")

    @functools.partial(
        pl.kernel,
        mesh=mesh,
        out_type=(
            jax.ShapeDtypeStruct((N1, H), jnp.float32),
            jax.ShapeDtypeStruct((N2, H), jnp.float32),
            jax.ShapeDtypeStruct((N3, H), jnp.float32),
        ),
        scratch_types=[
            pltpu.VMEM((P1,), jnp.int32),
            pltpu.VMEM((P2,), jnp.int32),
            pltpu.VMEM((P3,), jnp.int32),
            pltpu.VMEM((NBUF, CH, H), jnp.float32),
            pltpu.SemaphoreType.DMA((NBUF,)),
            pltpu.SemaphoreType.DMA((NBUF,)),
        ],
    )
    def k(idx1, idx2, idx3, t1, t2, t3, o1, o2, o3,
          idx1_v, idx2_v, idx3_v, rows_v, gsem, wsem):
        wid = lax.axis_index("s") * NC + lax.axis_index("c")

        def run(idx_hbm, idx_v, table_hbm, out_hbm, per_worker):
            n = per_worker // CH
            base = wid * per_worker
            pltpu.sync_copy(idx_hbm.at[pl.ds(base, per_worker)], idx_v)

            def gth(c, b):
                return pltpu.make_async_copy(
                    table_hbm.at[idx_v.at[pl.ds(c * CH, CH)]],
                    rows_v.at[b], gsem.at[b])

            def wb(c, b):
                return pltpu.make_async_copy(
                    rows_v.at[b], out_hbm.at[pl.ds(base + c * CH, CH)],
                    wsem.at[b])

            for b in range(NBUF):
                gth(b, b).start()

            m4 = ((n - NBUF) // NBUF) * NBUF

            def body(i, _):
                for b in range(NBUF):
                    c = i * NBUF + b
                    gth(c, b).wait()
                    wb(c, b).start()
                    wb(c, b).wait()
                    gth(c + NBUF, b).start()
                return 0

            lax.fori_loop(0, m4 // NBUF, body, 0)

            for cc in range(m4, n):
                b = cc % NBUF
                gth(cc, b).wait()
                wb(cc, b).start()
                wb(cc, b).wait()
                if cc + NBUF < n:
                    gth(cc + NBUF, b).start()

        run(idx1, idx1_v, t1, o1, P1)
        run(idx2, idx2_v, t2, o2, P2)
        run(idx3, idx3_v, t3, o3, P3)

    return k


_LB = 8
_MB = 1024
_NK1, _NK2, _NK3 = L1P // _LB, L2 // _LB, L3P // _LB
_NK = _NK1 + _NK2 + _NK3


def _tc_body(g1, g2, g3, w1, w2, w3, bi, wf, bf, out_ref, acc_ref):
    k = pl.program_id(1)

    @pl.when(k == 0)
    def _():
        acc_ref[...] = jnp.broadcast_to(bi[...], (_MB, NHID))

    def seg(g, w):
        for i in range(_LB):
            acc_ref[...] += jnp.dot(
                g[:, i, :].astype(jnp.bfloat16), w[i * H : (i + 1) * H, :],
                preferred_element_type=jnp.float32)

    @pl.when(k < _NK1)
    def _():
        seg(g1[...], w1[...])

    @pl.when((k >= _NK1) & (k < _NK1 + _NK2))
    def _():
        seg(g2[...], w2[...])

    @pl.when(k >= _NK1 + _NK2)
    def _():
        seg(g3[...], w3[...])

    @pl.when(k == _NK - 1)
    def _():
        r = acc_ref[...] * wf[...]
        out_ref[...] = jnp.sum(r, axis=1, keepdims=True) + bf[0, 0]


def _tc_mlp(g1, g2, g3, w1, w2, w3, b_inter, w_final, b_final):
    def g_spec(lo, nk):
        return pl.BlockSpec(
            (_MB, _LB, H),
            lambda m, k: (m, jnp.clip(k - lo, 0, nk - 1), 0),
        )

    def w_spec(lo, nk):
        return pl.BlockSpec(
            (_LB * H, NHID),
            lambda m, k: (jnp.clip(k - lo, 0, nk - 1), 0),
        )

    return pl.pallas_call(
        _tc_body,
        grid=(B // _MB, _NK),
        in_specs=[
            g_spec(0, _NK1),
            g_spec(_NK1, _NK2),
            g_spec(_NK1 + _NK2, _NK3),
            w_spec(0, _NK1),
            w_spec(_NK1, _NK2),
            w_spec(_NK1 + _NK2, _NK3),
            pl.BlockSpec((1, NHID), lambda m, k: (0, 0)),
            pl.BlockSpec((1, NHID), lambda m, k: (0, 0)),
            pl.BlockSpec(memory_space=pltpu.SMEM),
        ],
        out_specs=pl.BlockSpec((_MB, 1), lambda m, k: (m, 0)),
        out_shape=jax.ShapeDtypeStruct((B, 1), jnp.float32),
        scratch_shapes=[pltpu.VMEM((_MB, NHID), jnp.float32)],
    )(g1, g2, g3, w1, w2, w3, b_inter, w_final, b_final)


def _rearrange_w(w_seg, lt, lpad):
    wt = w_seg.reshape(NHID, H, lt).transpose(2, 1, 0).reshape(lt * H, NHID)
    if lpad != lt:
        wt = jnp.concatenate(
            [wt, jnp.zeros(((lpad - lt) * H, NHID), wt.dtype)])
    return wt


def kernel(input1, input2, input3, title_emb, full_emb, cat_emb,
           W_inter, b_inter, W_final, b_final):
    pad1 = (jnp.arange(B * (L1P - L1), dtype=jnp.int32)
            .reshape(B, L1P - L1) % 100000)
    pad3 = (jnp.arange(B * (L3P - L3), dtype=jnp.int32)
            .reshape(B, L3P - L3) % 26)
    idx1 = jnp.concatenate([input1.astype(jnp.int32), pad1], axis=1).reshape(-1)
    idx2 = input2.reshape(-1).astype(jnp.int32)
    idx3 = jnp.concatenate([input3.astype(jnp.int32), pad3], axis=1).reshape(-1)

    g1, g2, g3 = _sc_gather()(idx1, idx2, idx3, title_emb, full_emb, cat_emb)

    wb = W_inter.astype(jnp.bfloat16)
    w1 = _rearrange_w(wb[:, : H * L1], L1, L1P)
    w2 = _rearrange_w(wb[:, H * L1 : H * (L1 + L2)], L2, L2)
    w3 = _rearrange_w(wb[:, H * (L1 + L2) :], L3, L3P)

    return _tc_mlp(
        g1.reshape(B, L1P, H), g2.reshape(B, L2, H), g3.reshape(B, L3P, H),
        w1, w2, w3,
        b_inter.reshape(1, NHID),
        W_final.reshape(1, NHID),
        b_final.reshape(1, 1).astype(jnp.float32),
    )


# speedup vs baseline: 2.7068x; 1.0671x over previous
import functools

import jax
import jax.numpy as jnp
from jax import lax
from jax.experimental import pallas as pl
from jax.experimental.pallas import tpu as pltpu
from jax.experimental.pallas import tpu_sc as plsc

B = 4096
L1, L2, L3 = 20, 200, 26
L1P, L3P = 24, 32
H = 128
NHID = 256

NC, NS = 2, 16
NW = NC * NS
CH = 128
NBUF = 4

def _sc_gather(nb):
    n1, n2, n3 = nb * L1P, nb * L2, nb * L3P
    p1, p2, p3 = n1 // NW, n2 // NW, n3 // NW
    mesh = plsc.VectorSubcoreMesh(core_axis_name="c", subcore_axis_name="s")

    @functools.partial(
        pl.kernel,
        mesh=mesh,
        out_type=(
            jax.ShapeDtypeStruct((n1, H), jnp.float32),
            jax.ShapeDtypeStruct((n2, H), jnp.float32),
            jax.ShapeDtypeStruct((n3, H), jnp.float32),
        ),
        scratch_types=[
            pltpu.VMEM((p1,), jnp.int32),
            pltpu.VMEM((p2,), jnp.int32),
            pltpu.VMEM((p3,), jnp.int32),
            pltpu.VMEM((NBUF, CH, H), jnp.float32),
            pltpu.SemaphoreType.DMA((NBUF,)),
            pltpu.SemaphoreType.DMA((NBUF,)),
        ],
    )
    def k(idx1, idx2, idx3, t1, t2, t3, o1, o2, o3,
          idx1_v, idx2_v, idx3_v, rows_v, gsem, wsem):
        wid = lax.axis_index("s") * NC + lax.axis_index("c")

        def run(idx_hbm, idx_v, table_hbm, out_hbm, per_worker):
            n = per_worker // CH
            base = wid * per_worker
            pltpu.sync_copy(idx_hbm.at[pl.ds(base, per_worker)], idx_v)

            def gth(c, b):
                return pltpu.make_async_copy(
                    table_hbm.at[idx_v.at[pl.ds(c * CH, CH)]],
                    rows_v.at[b], gsem.at[b])

            def wb(c, b):
                return pltpu.make_async_copy(
                    rows_v.at[b], out_hbm.at[pl.ds(base + c * CH, CH)],
                    wsem.at[b])

            for b in range(NBUF):
                gth(b, b).start()

            m4 = ((n - NBUF) // NBUF) * NBUF

            def body(i, _):
                for b in range(NBUF):
                    c = i * NBUF + b
                    gth(c, b).wait()
                    wb(c, b).start()
                    wb(c, b).wait()
                    gth(c + NBUF, b).start()
                return 0

            lax.fori_loop(0, m4 // NBUF, body, 0)

            for cc in range(m4, n):
                b = cc % NBUF
                gth(cc, b).wait()
                wb(cc, b).start()
                wb(cc, b).wait()
                if cc + NBUF < n:
                    gth(cc + NBUF, b).start()

        run(idx1, idx1_v, t1, o1, p1)
        run(idx2, idx2_v, t2, o2, p2)
        run(idx3, idx3_v, t3, o3, p3)

    return k


_LB = 8
_MB = 1024
_NK1, _NK2, _NK3 = L1P // _LB, L2 // _LB, L3P // _LB
_NK = _NK1 + _NK2 + _NK3


def _tc_body(g1, g2, g3, w1, w2, w3, bi, wf, bf, out_ref, acc_ref):
    k = pl.program_id(1)

    @pl.when(k == 0)
    def _():
        acc_ref[...] = jnp.broadcast_to(bi[...], (_MB, NHID))

    def seg(g, w):
        for i in range(_LB):
            acc_ref[...] += jnp.dot(
                g[:, i, :].astype(jnp.bfloat16), w[i * H : (i + 1) * H, :],
                preferred_element_type=jnp.float32)

    @pl.when(k < _NK1)
    def _():
        seg(g1[...], w1[...])

    @pl.when((k >= _NK1) & (k < _NK1 + _NK2))
    def _():
        seg(g2[...], w2[...])

    @pl.when(k >= _NK1 + _NK2)
    def _():
        seg(g3[...], w3[...])

    @pl.when(k == _NK - 1)
    def _():
        r = acc_ref[...] * wf[...]
        out_ref[...] = jnp.sum(r, axis=1, keepdims=True) + bf[0, 0]


def _tc_mlp(nb, g1, g2, g3, w1, w2, w3, b_inter, w_final, b_final):
    def g_spec(lo, nk):
        return pl.BlockSpec(
            (_MB, _LB, H),
            lambda m, k: (m, jnp.clip(k - lo, 0, nk - 1), 0),
        )

    def w_spec(lo, nk):
        return pl.BlockSpec(
            (_LB * H, NHID),
            lambda m, k: (jnp.clip(k - lo, 0, nk - 1), 0),
        )

    return pl.pallas_call(
        _tc_body,
        grid=(nb // _MB, _NK),
        in_specs=[
            g_spec(0, _NK1),
            g_spec(_NK1, _NK2),
            g_spec(_NK1 + _NK2, _NK3),
            w_spec(0, _NK1),
            w_spec(_NK1, _NK2),
            w_spec(_NK1 + _NK2, _NK3),
            pl.BlockSpec((1, NHID), lambda m, k: (0, 0)),
            pl.BlockSpec((1, NHID), lambda m, k: (0, 0)),
            pl.BlockSpec(memory_space=pltpu.SMEM),
        ],
        out_specs=pl.BlockSpec((_MB, 1), lambda m, k: (m, 0)),
        out_shape=jax.ShapeDtypeStruct((nb, 1), jnp.float32),
        scratch_shapes=[pltpu.VMEM((_MB, NHID), jnp.float32)],
    )(g1, g2, g3, w1, w2, w3, b_inter, w_final, b_final)


def _rearrange_w(w_seg, lt, lpad):
    wt = w_seg.reshape(NHID, H, lt).transpose(2, 1, 0).reshape(lt * H, NHID)
    if lpad != lt:
        wt = jnp.concatenate(
            [wt, jnp.zeros(((lpad - lt) * H, NHID), wt.dtype)])
    return wt


_NSPLIT = 2


def kernel(input1, input2, input3, title_emb, full_emb, cat_emb,
           W_inter, b_inter, W_final, b_final):
    pad1 = (jnp.arange(B * (L1P - L1), dtype=jnp.int32)
            .reshape(B, L1P - L1) % 100000)
    pad3 = (jnp.arange(B * (L3P - L3), dtype=jnp.int32)
            .reshape(B, L3P - L3) % 26)
    idx1 = jnp.concatenate([input1.astype(jnp.int32), pad1], axis=1).reshape(-1)
    idx2 = input2.reshape(-1).astype(jnp.int32)
    idx3 = jnp.concatenate([input3.astype(jnp.int32), pad3], axis=1).reshape(-1)

    wb = W_inter.astype(jnp.bfloat16)
    w1 = _rearrange_w(wb[:, : H * L1], L1, L1P)
    w2 = _rearrange_w(wb[:, H * L1 : H * (L1 + L2)], L2, L2)
    w3 = _rearrange_w(wb[:, H * (L1 + L2) :], L3, L3P)
    bi = b_inter.reshape(1, NHID)
    wf = W_final.reshape(1, NHID)
    bf = b_final.reshape(1, 1).astype(jnp.float32)

    nb = B // _NSPLIT
    gather = _sc_gather(nb)
    outs = []
    for s in range(_NSPLIT):
        sl = slice(s * nb * L1P, (s + 1) * nb * L1P)
        s2 = slice(s * nb * L2, (s + 1) * nb * L2)
        s3 = slice(s * nb * L3P, (s + 1) * nb * L3P)
        g1, g2, g3 = gather(idx1[sl], idx2[s2], idx3[s3],
                            title_emb, full_emb, cat_emb)
        outs.append(_tc_mlp(
            nb,
            g1.reshape(nb, L1P, H), g2.reshape(nb, L2, H),
            g3.reshape(nb, L3P, H),
            w1, w2, w3, bi, wf, bf,
        ))
    return jnp.concatenate(outs, axis=0)


# speedup vs baseline: 2.8057x; 1.0365x over previous
import functools

import jax
import jax.numpy as jnp
from jax import lax
from jax.experimental import pallas as pl
from jax.experimental.pallas import tpu as pltpu
from jax.experimental.pallas import tpu_sc as plsc

B = 4096
L1, L2, L3 = 20, 200, 26
L1P, L3P = 24, 32
H = 128
NHID = 256

NC, NS = 2, 16
NW = NC * NS
CH = 128
NBUF = 4

def _sc_gather(nb):
    n1, n2, n3 = nb * L1P, nb * L2, nb * L3P
    p1, p2, p3 = n1 // NW, n2 // NW, n3 // NW
    mesh = plsc.VectorSubcoreMesh(core_axis_name="c", subcore_axis_name="s")

    @functools.partial(
        pl.kernel,
        mesh=mesh,
        out_type=(
            jax.ShapeDtypeStruct((n1, H), jnp.float32),
            jax.ShapeDtypeStruct((n2, H), jnp.float32),
            jax.ShapeDtypeStruct((n3, H), jnp.float32),
        ),
        scratch_types=[
            pltpu.VMEM((p1,), jnp.int32),
            pltpu.VMEM((p2,), jnp.int32),
            pltpu.VMEM((p3,), jnp.int32),
            pltpu.VMEM((NBUF, CH, H), jnp.float32),
            pltpu.SemaphoreType.DMA((NBUF,)),
            pltpu.SemaphoreType.DMA((NBUF,)),
        ],
    )
    def k(idx1, idx2, idx3, t1, t2, t3, o1, o2, o3,
          idx1_v, idx2_v, idx3_v, rows_v, gsem, wsem):
        wid = lax.axis_index("s") * NC + lax.axis_index("c")

        def run(idx_hbm, idx_v, table_hbm, out_hbm, per_worker):
            n = per_worker // CH
            base = wid * per_worker
            pltpu.sync_copy(idx_hbm.at[pl.ds(base, per_worker)], idx_v)

            def gth(c, b):
                return pltpu.make_async_copy(
                    table_hbm.at[idx_v.at[pl.ds(c * CH, CH)]],
                    rows_v.at[b], gsem.at[b])

            def wb(c, b):
                return pltpu.make_async_copy(
                    rows_v.at[b], out_hbm.at[pl.ds(base + c * CH, CH)],
                    wsem.at[b])

            for b in range(NBUF):
                gth(b, b).start()

            m4 = ((n - NBUF) // NBUF) * NBUF

            def body(i, _):
                for b in range(NBUF):
                    c = i * NBUF + b
                    gth(c, b).wait()
                    wb(c, b).start()
                    wb(c, b).wait()
                    gth(c + NBUF, b).start()
                return 0

            lax.fori_loop(0, m4 // NBUF, body, 0)

            for cc in range(m4, n):
                b = cc % NBUF
                gth(cc, b).wait()
                wb(cc, b).start()
                wb(cc, b).wait()
                if cc + NBUF < n:
                    gth(cc + NBUF, b).start()

        run(idx1, idx1_v, t1, o1, p1)
        run(idx2, idx2_v, t2, o2, p2)
        run(idx3, idx3_v, t3, o3, p3)

    return k


_LB = 8
_MB = 1024
_NK1, _NK2, _NK3 = L1P // _LB, L2 // _LB, L3P // _LB
_NK = _NK1 + _NK2 + _NK3


def _tc_body(g1, g2, g3, w1, w2, w3, bi, wf, bf, out_ref, acc_ref):
    k = pl.program_id(1)

    @pl.when(k == 0)
    def _():
        acc_ref[...] = jnp.broadcast_to(bi[...], (_MB, NHID))

    def seg(g, w):
        for i in range(_LB):
            acc_ref[...] += jnp.dot(
                g[:, i, :].astype(jnp.bfloat16), w[i * H : (i + 1) * H, :],
                preferred_element_type=jnp.float32)

    @pl.when(k < _NK1)
    def _():
        seg(g1[...], w1[...])

    @pl.when((k >= _NK1) & (k < _NK1 + _NK2))
    def _():
        seg(g2[...], w2[...])

    @pl.when(k >= _NK1 + _NK2)
    def _():
        seg(g3[...], w3[...])

    @pl.when(k == _NK - 1)
    def _():
        r = acc_ref[...] * wf[...]
        out_ref[...] = jnp.sum(r, axis=1, keepdims=True) + bf[0, 0]


def _tc_mlp(nb, g1, g2, g3, w1, w2, w3, b_inter, w_final, b_final):
    def g_spec(lo, nk):
        return pl.BlockSpec(
            (_MB, _LB, H),
            lambda m, k: (m, jnp.clip(k - lo, 0, nk - 1), 0),
        )

    def w_spec(lo, nk):
        return pl.BlockSpec(
            (_LB * H, NHID),
            lambda m, k: (jnp.clip(k - lo, 0, nk - 1), 0),
        )

    return pl.pallas_call(
        _tc_body,
        grid=(nb // _MB, _NK),
        in_specs=[
            g_spec(0, _NK1),
            g_spec(_NK1, _NK2),
            g_spec(_NK1 + _NK2, _NK3),
            w_spec(0, _NK1),
            w_spec(_NK1, _NK2),
            w_spec(_NK1 + _NK2, _NK3),
            pl.BlockSpec((1, NHID), lambda m, k: (0, 0)),
            pl.BlockSpec((1, NHID), lambda m, k: (0, 0)),
            pl.BlockSpec(memory_space=pltpu.SMEM),
        ],
        out_specs=pl.BlockSpec((_MB, 1), lambda m, k: (m, 0)),
        out_shape=jax.ShapeDtypeStruct((nb, 1), jnp.float32),
        scratch_shapes=[pltpu.VMEM((_MB, NHID), jnp.float32)],
    )(g1, g2, g3, w1, w2, w3, b_inter, w_final, b_final)


def _rearrange_w(w_seg, lt, lpad):
    wt = w_seg.reshape(NHID, H, lt).transpose(2, 1, 0).reshape(lt * H, NHID)
    if lpad != lt:
        wt = jnp.concatenate(
            [wt, jnp.zeros(((lpad - lt) * H, NHID), wt.dtype)])
    return wt


_NSPLIT = 4


def kernel(input1, input2, input3, title_emb, full_emb, cat_emb,
           W_inter, b_inter, W_final, b_final):
    pad1 = (jnp.arange(B * (L1P - L1), dtype=jnp.int32)
            .reshape(B, L1P - L1) % 100000)
    pad3 = (jnp.arange(B * (L3P - L3), dtype=jnp.int32)
            .reshape(B, L3P - L3) % 26)
    idx1 = jnp.concatenate([input1.astype(jnp.int32), pad1], axis=1).reshape(-1)
    idx2 = input2.reshape(-1).astype(jnp.int32)
    idx3 = jnp.concatenate([input3.astype(jnp.int32), pad3], axis=1).reshape(-1)

    wb = W_inter.astype(jnp.bfloat16)
    w1 = _rearrange_w(wb[:, : H * L1], L1, L1P)
    w2 = _rearrange_w(wb[:, H * L1 : H * (L1 + L2)], L2, L2)
    w3 = _rearrange_w(wb[:, H * (L1 + L2) :], L3, L3P)
    bi = b_inter.reshape(1, NHID)
    wf = W_final.reshape(1, NHID)
    bf = b_final.reshape(1, 1).astype(jnp.float32)

    nb = B // _NSPLIT
    gather = _sc_gather(nb)
    outs = []
    for s in range(_NSPLIT):
        sl = slice(s * nb * L1P, (s + 1) * nb * L1P)
        s2 = slice(s * nb * L2, (s + 1) * nb * L2)
        s3 = slice(s * nb * L3P, (s + 1) * nb * L3P)
        g1, g2, g3 = gather(idx1[sl], idx2[s2], idx3[s3],
                            title_emb, full_emb, cat_emb)
        outs.append(_tc_mlp(
            nb,
            g1.reshape(nb, L1P, H), g2.reshape(nb, L2, H),
            g3.reshape(nb, L3P, H),
            w1, w2, w3, bi, wf, bf,
        ))
    return jnp.concatenate(outs, axis=0)
